# Initial kernel scaffold; baseline (speedup 1.0000x reference)
import jax
import jax.numpy as jnp
from jax.experimental import pallas as pl


def kernel(x, Wg, bg, W1, b1, W2, b2):
    raise NotImplementedError("write your pallas kernel here")


# speedup vs baseline: 2.9171x; 2.9171x over previous
import functools

import jax
import jax.numpy as jnp
from jax import lax
from jax.experimental import pallas as pl
from jax.experimental.pallas import tpu as pltpu
from jax.experimental.pallas import tpu_sc as plsc

E = 64
K = 4
T = 128
NB = 512

_NC, _NS = 2, 16
_NW = _NC * _NS


def _routing_body(x_ref, wg_ref, bg_ref, logits_ref, wtop_ref, itop_ref):
    x = x_ref[...]
    wg = wg_ref[...]
    logits = lax.dot_general(
        x, wg, (((1,), (1,)), ((), ())),
        preferred_element_type=jnp.float32,
    ) + bg_ref[...]
    logits_ref[...] = logits

    iota = lax.broadcasted_iota(jnp.int32, logits.shape, 1)
    cur = logits
    sel_l, sel_i = [], []
    for _ in range(K):
        mk = jnp.max(cur, axis=1, keepdims=True)
        ik = jnp.min(jnp.where(cur == mk, iota, E), axis=1, keepdims=True)
        sel_l.append(mk)
        sel_i.append(ik)
        cur = jnp.where(iota == ik, -1e30, cur)
    m0 = sel_l[0]
    exps = [jnp.exp(l - m0) for l in sel_l]
    denom = exps[0] + exps[1] + exps[2] + exps[3]
    wtop_ref[...] = jnp.concatenate([e / denom for e in exps], axis=1)
    itop_ref[...] = jnp.concatenate(sel_i, axis=1)


def _routing(x2d, Wg, bg):
    n, d = x2d.shape
    grid = (n // NB,)
    return pl.pallas_call(
        _routing_body,
        grid=grid,
        in_specs=[
            pl.BlockSpec((NB, d), lambda i: (i, 0)),
            pl.BlockSpec((E, d), lambda i: (0, 0)),
            pl.BlockSpec((1, E), lambda i: (0, 0)),
        ],
        out_specs=[
            pl.BlockSpec((NB, E), lambda i: (i, 0)),
            pl.BlockSpec((NB, K), lambda i: (i, 0)),
            pl.BlockSpec((NB, K), lambda i: (i, 0)),
        ],
        out_shape=[
            jax.ShapeDtypeStruct((n, E), jnp.float32),
            jax.ShapeDtypeStruct((n, K), jnp.float32),
            jax.ShapeDtypeStruct((n, K), jnp.int32),
        ],
    )(x2d, Wg, bg.reshape(1, E))


def _dispatch_metadata(itop, wtop, n_tiles_max):
    nk = itop.shape[0] * K
    i32 = jnp.int32
    iflat = itop.reshape(-1).astype(i32)
    wflat = wtop.reshape(-1)
    p_tok = (jnp.arange(nk, dtype=i32) // K)
    oh = (iflat[:, None] == jnp.arange(E, dtype=i32)[None, :]).astype(i32)
    csum = jnp.cumsum(oh, axis=0)
    rank = jnp.take_along_axis(csum, iflat[:, None], axis=1)[:, 0] - 1
    counts = csum[-1]
    tiles_e = (counts + T - 1) // T
    pad_sz = tiles_e * T
    pad_off = jnp.cumsum(pad_sz) - pad_sz
    slot = pad_off[iflat] + rank
    total_tiles = jnp.sum(tiles_e)
    tile_cum = jnp.cumsum(tiles_e)
    gidx = jnp.arange(n_tiles_max, dtype=i32)
    tile_expert = jnp.minimum(
        jnp.searchsorted(tile_cum, gidx, side="right").astype(i32), E - 1)
    tile_valid = (gidx < total_tiles).astype(i32)
    tile_blk = jnp.where(tile_valid == 1, gidx, total_tiles - 1).astype(i32)
    n_rows = n_tiles_max * T
    tok_pad = jnp.zeros((n_rows,), i32).at[slot].set(p_tok)
    w_pad = jnp.zeros((n_rows,), jnp.float32).at[slot].set(wflat)
    return slot, tok_pad, w_pad, tile_expert, tile_blk, tile_valid


def _sc_gather(src, idx, chunk):
    m = idx.shape[0]
    d = src.shape[1]
    per_w = m // _NW
    n_iter = per_w // chunk
    mesh = plsc.VectorSubcoreMesh(core_axis_name="c", subcore_axis_name="s")

    @functools.partial(
        pl.kernel,
        out_type=jax.ShapeDtypeStruct((m, d), src.dtype),
        mesh=mesh,
        scratch_types=[
            pltpu.VMEM((chunk,), jnp.int32),
            pltpu.VMEM((chunk, d), src.dtype),
            pltpu.SemaphoreType.DMA,
        ],
    )
    def gather_k(src_hbm, idx_hbm, out_hbm, idx_v, rows_v, sem):
        wid = lax.axis_index("s") * _NC + lax.axis_index("c")
        base = wid * per_w
        for i in range(n_iter):
            off = base + i * chunk
            pltpu.sync_copy(idx_hbm.at[pl.ds(off, chunk)], idx_v)
            pltpu.async_copy(src_hbm.at[idx_v], rows_v, sem).wait()
            pltpu.sync_copy(rows_v, out_hbm.at[pl.ds(off, chunk)])

    return gather_k(src, idx)


def _ffn_body(blk_ref, exp_ref, val_ref, xg_ref, w1_ref, b1_ref, w2_ref,
              b2_ref, ws_ref, out_ref):
    g = pl.program_id(0)

    @pl.when(val_ref[g] == 1)
    def _():
        x = xg_ref[...]
        h = lax.dot_general(
            x, w1_ref[0], (((1,), (1,)), ((), ())),
            preferred_element_type=jnp.float32)
        h = h + b1_ref[0]
        h = 0.5 * h * (1.0 + lax.erf(h * 0.7071067811865476))
        y = lax.dot_general(
            h, w2_ref[0], (((1,), (1,)), ((), ())),
            preferred_element_type=jnp.float32)
        y = y + b2_ref[0]
        out_ref[...] = y * ws_ref[...]


def _grouped_ffn(xg, W1, b1, W2, b2, w_pad, tile_blk, tile_expert, tile_valid,
                 n_tiles_max):
    d = W1.shape[2]
    h = W1.shape[1]
    n_rows = n_tiles_max * T
    grid_spec = pltpu.PrefetchScalarGridSpec(
        num_scalar_prefetch=3,
        grid=(n_tiles_max,),
        in_specs=[
            pl.BlockSpec((T, d), lambda g, blk, exp, val: (blk[g], 0)),
            pl.BlockSpec((1, h, d), lambda g, blk, exp, val: (exp[g], 0, 0)),
            pl.BlockSpec((1, 1, h), lambda g, blk, exp, val: (exp[g], 0, 0)),
            pl.BlockSpec((1, d, h), lambda g, blk, exp, val: (exp[g], 0, 0)),
            pl.BlockSpec((1, 1, d), lambda g, blk, exp, val: (exp[g], 0, 0)),
            pl.BlockSpec((T, 1), lambda g, blk, exp, val: (blk[g], 0)),
        ],
        out_specs=pl.BlockSpec((T, d), lambda g, blk, exp, val: (blk[g], 0)),
    )
    return pl.pallas_call(
        _ffn_body,
        grid_spec=grid_spec,
        out_shape=jax.ShapeDtypeStruct((n_rows, d), jnp.float32),
    )(tile_blk, tile_expert, tile_valid, xg, W1, b1.reshape(E, 1, h), W2,
      b2.reshape(E, 1, d), w_pad.reshape(n_rows, 1))


def _combine_body(yp_ref, out_ref):
    out_ref[...] = jnp.sum(yp_ref[...], axis=1)


def _combine(yp, n, d):
    blk = 256
    y3 = yp.reshape(n, K, d)
    return pl.pallas_call(
        _combine_body,
        grid=(n // blk,),
        in_specs=[pl.BlockSpec((blk, K, d), lambda i: (i, 0, 0))],
        out_specs=pl.BlockSpec((blk, d), lambda i: (i, 0)),
        out_shape=jax.ShapeDtypeStruct((n, d), jnp.float32),
    )(y3)


def kernel(x, Wg, bg, W1, b1, W2, b2):
    b, s, d = x.shape
    n = b * s
    nk = n * K
    n_tiles_max = nk // T + E
    x2d = x.reshape(n, d)

    logits, wtop, itop = _routing(x2d, Wg, bg)
    slot, tok_pad, w_pad, tile_expert, tile_blk, tile_valid = (
        _dispatch_metadata(itop, wtop, n_tiles_max))

    xg = _sc_gather(x2d, tok_pad, chunk=128)
    yg = _grouped_ffn(xg, W1, b1, W2, b2, w_pad, tile_blk, tile_expert,
                      tile_valid, n_tiles_max)
    yp = _sc_gather(yg, slot, chunk=128)
    final = _combine(yp, n, d)
    return final.reshape(b, s, d), logits


# speedup vs baseline: 4.6283x; 1.5866x over previous
import functools

import jax
import jax.numpy as jnp
from jax import lax
from jax.experimental import pallas as pl
from jax.experimental.pallas import tpu as pltpu
from jax.experimental.pallas import tpu_sc as plsc

E = 64
K = 4
T = 128
NB = 512

_NC, _NS = 2, 16
_NW = _NC * _NS


def _routing_body(x_ref, wg_ref, bg_ref, logits_ref, wtop_ref, itop_ref):
    x = x_ref[...]
    wg = wg_ref[...]
    logits = lax.dot_general(
        x, wg, (((1,), (1,)), ((), ())),
        preferred_element_type=jnp.float32,
    ) + bg_ref[...]
    logits_ref[...] = logits

    iota = lax.broadcasted_iota(jnp.int32, logits.shape, 1)
    cur = logits
    sel_l, sel_i = [], []
    for _ in range(K):
        mk = jnp.max(cur, axis=1, keepdims=True)
        ik = jnp.min(jnp.where(cur == mk, iota, E), axis=1, keepdims=True)
        sel_l.append(mk)
        sel_i.append(ik)
        cur = jnp.where(iota == ik, -1e30, cur)
    m0 = sel_l[0]
    exps = [jnp.exp(l - m0) for l in sel_l]
    denom = exps[0] + exps[1] + exps[2] + exps[3]
    wtop_ref[...] = jnp.concatenate([e / denom for e in exps], axis=1)
    itop_ref[...] = jnp.concatenate(sel_i, axis=1)


def _routing(x2d, Wg, bg):
    n, d = x2d.shape
    grid = (n // NB,)
    return pl.pallas_call(
        _routing_body,
        grid=grid,
        in_specs=[
            pl.BlockSpec((NB, d), lambda i: (i, 0)),
            pl.BlockSpec((E, d), lambda i: (0, 0)),
            pl.BlockSpec((1, E), lambda i: (0, 0)),
        ],
        out_specs=[
            pl.BlockSpec((NB, E), lambda i: (i, 0)),
            pl.BlockSpec((NB, K), lambda i: (i, 0)),
            pl.BlockSpec((NB, K), lambda i: (i, 0)),
        ],
        out_shape=[
            jax.ShapeDtypeStruct((n, E), jnp.float32),
            jax.ShapeDtypeStruct((n, K), jnp.float32),
            jax.ShapeDtypeStruct((n, K), jnp.int32),
        ],
    )(x2d, Wg, bg.reshape(1, E))


def _dispatch_metadata(itop, wtop, n_tiles_max):
    nk = itop.shape[0] * K
    i32 = jnp.int32
    iflat = itop.reshape(-1).astype(i32)
    wflat = wtop.reshape(-1)
    p_tok = (jnp.arange(nk, dtype=i32) // K)
    oh = (iflat[:, None] == jnp.arange(E, dtype=i32)[None, :]).astype(i32)
    csum = jnp.cumsum(oh, axis=0)
    rank = jnp.take_along_axis(csum, iflat[:, None], axis=1)[:, 0] - 1
    counts = csum[-1]
    tiles_e = (counts + T - 1) // T
    pad_sz = tiles_e * T
    pad_off = jnp.cumsum(pad_sz) - pad_sz
    slot = pad_off[iflat] + rank
    total_tiles = jnp.sum(tiles_e)
    tile_cum = jnp.cumsum(tiles_e)
    gidx = jnp.arange(n_tiles_max, dtype=i32)
    tile_expert = jnp.minimum(
        jnp.searchsorted(tile_cum, gidx, side="right").astype(i32), E - 1)
    tile_valid = (gidx < total_tiles).astype(i32)
    tile_blk = jnp.where(tile_valid == 1, gidx, total_tiles - 1).astype(i32)
    n_rows = n_tiles_max * T
    n_tok = itop.shape[0]
    tok_pad = (jnp.arange(n_rows, dtype=i32) % n_tok).at[slot].set(p_tok)
    w_pad = jnp.zeros((n_rows,), jnp.float32).at[slot].set(wflat)
    return slot, tok_pad, w_pad, tile_expert, tile_blk, tile_valid


def _sc_gather(src, idx, chunk):
    m = idx.shape[0]
    d = src.shape[1]
    per_w = m // _NW
    n_iter = per_w // chunk
    mesh = plsc.VectorSubcoreMesh(core_axis_name="c", subcore_axis_name="s")

    @functools.partial(
        pl.kernel,
        out_type=jax.ShapeDtypeStruct((m, d), src.dtype),
        mesh=mesh,
        scratch_types=[
            pltpu.VMEM((chunk,), jnp.int32),
            pltpu.VMEM((chunk, d), src.dtype),
            pltpu.SemaphoreType.DMA,
        ],
    )
    def gather_k(src_hbm, idx_hbm, out_hbm, idx_v, rows_v, sem):
        wid = lax.axis_index("s") * _NC + lax.axis_index("c")
        base = wid * per_w
        for i in range(n_iter):
            off = base + i * chunk
            pltpu.sync_copy(idx_hbm.at[pl.ds(off, chunk)], idx_v)
            pltpu.async_copy(src_hbm.at[idx_v], rows_v, sem).wait()
            pltpu.sync_copy(rows_v, out_hbm.at[pl.ds(off, chunk)])

    return gather_k(src, idx)


def _ffn_body(blk_ref, exp_ref, val_ref, xg_ref, w1_ref, b1_ref, w2_ref,
              b2_ref, ws_ref, out_ref):
    g = pl.program_id(0)

    @pl.when(val_ref[g] == 1)
    def _():
        x = xg_ref[...]
        h = lax.dot_general(
            x, w1_ref[0], (((1,), (1,)), ((), ())),
            preferred_element_type=jnp.float32)
        h = h + b1_ref[0]
        h = 0.5 * h * (1.0 + lax.erf(h * 0.7071067811865476))
        y = lax.dot_general(
            h, w2_ref[0], (((1,), (1,)), ((), ())),
            preferred_element_type=jnp.float32)
        y = y + b2_ref[0]
        out_ref[...] = y * ws_ref[...]


def _grouped_ffn(xg, W1, b1, W2, b2, w_pad, tile_blk, tile_expert, tile_valid,
                 n_tiles_max):
    d = W1.shape[2]
    h = W1.shape[1]
    n_rows = n_tiles_max * T
    grid_spec = pltpu.PrefetchScalarGridSpec(
        num_scalar_prefetch=3,
        grid=(n_tiles_max,),
        in_specs=[
            pl.BlockSpec((T, d), lambda g, blk, exp, val: (blk[g], 0)),
            pl.BlockSpec((1, h, d), lambda g, blk, exp, val: (exp[g], 0, 0)),
            pl.BlockSpec((1, 1, h), lambda g, blk, exp, val: (exp[g], 0, 0)),
            pl.BlockSpec((1, d, h), lambda g, blk, exp, val: (exp[g], 0, 0)),
            pl.BlockSpec((1, 1, d), lambda g, blk, exp, val: (exp[g], 0, 0)),
            pl.BlockSpec((T, 1), lambda g, blk, exp, val: (blk[g], 0)),
        ],
        out_specs=pl.BlockSpec((T, d), lambda g, blk, exp, val: (blk[g], 0)),
    )
    return pl.pallas_call(
        _ffn_body,
        grid_spec=grid_spec,
        out_shape=jax.ShapeDtypeStruct((n_rows, d), jnp.float32),
    )(tile_blk, tile_expert, tile_valid, xg, W1, b1.reshape(E, 1, h), W2,
      b2.reshape(E, 1, d), w_pad.reshape(n_rows, 1))


def _combine_body(yp_ref, out_ref):
    y = yp_ref[...]
    d = out_ref.shape[1]
    acc = y[:, 0:d] + y[:, d:2 * d]
    acc = acc + y[:, 2 * d:3 * d]
    out_ref[...] = acc + y[:, 3 * d:4 * d]


def _combine(yp, n, d):
    blk = 512
    y2 = yp.reshape(n, K * d)
    return pl.pallas_call(
        _combine_body,
        grid=(n // blk,),
        in_specs=[pl.BlockSpec((blk, K * d), lambda i: (i, 0))],
        out_specs=pl.BlockSpec((blk, d), lambda i: (i, 0)),
        out_shape=jax.ShapeDtypeStruct((n, d), jnp.float32),
    )(y2)


def kernel(x, Wg, bg, W1, b1, W2, b2):
    b, s, d = x.shape
    n = b * s
    nk = n * K
    n_tiles_max = nk // T + E
    x2d = x.reshape(n, d)

    logits, wtop, itop = _routing(x2d, Wg, bg)
    slot, tok_pad, w_pad, tile_expert, tile_blk, tile_valid = (
        _dispatch_metadata(itop, wtop, n_tiles_max))

    xg = _sc_gather(x2d, tok_pad, chunk=128)
    yg = _grouped_ffn(xg, W1, b1, W2, b2, w_pad, tile_blk, tile_expert,
                      tile_valid, n_tiles_max)
    yp = _sc_gather(yg, slot, chunk=128)
    final = _combine(yp, n, d)
    return final.reshape(b, s, d), logits


# speedup vs baseline: 4.7527x; 1.0269x over previous
import functools

import jax
import jax.numpy as jnp
from jax import lax
from jax.experimental import pallas as pl
from jax.experimental.pallas import tpu as pltpu
from jax.experimental.pallas import tpu_sc as plsc

E = 64
K = 4
T = 128
NB = 512

_NC, _NS = 2, 16
_NW = _NC * _NS


def _routing_body(x_ref, wg_ref, bg_ref, logits_ref, wtop_ref, itop_ref,
                  rank_ref, counts_ref, carry_ref):
    pid = pl.program_id(0)

    @pl.when(pid == 0)
    def _():
        carry_ref[...] = jnp.zeros_like(carry_ref)

    x = x_ref[...]
    wg = wg_ref[...]
    logits = lax.dot_general(
        x, wg, (((1,), (1,)), ((), ())),
        preferred_element_type=jnp.float32,
    ) + bg_ref[...]
    logits_ref[...] = logits

    iota = lax.broadcasted_iota(jnp.int32, logits.shape, 1)
    cur = logits
    sel_l, sel_i = [], []
    for _ in range(K):
        mk = jnp.max(cur, axis=1, keepdims=True)
        ik = jnp.min(jnp.where(cur == mk, iota, E), axis=1, keepdims=True)
        sel_l.append(mk)
        sel_i.append(ik)
        cur = jnp.where(iota == ik, -1e30, cur)
    m0 = sel_l[0]
    exps = [jnp.exp(l - m0) for l in sel_l]
    denom = exps[0] + exps[1] + exps[2] + exps[3]
    wtop_ref[...] = jnp.concatenate([e / denom for e in exps], axis=1)
    itop_ref[...] = jnp.concatenate(sel_i, axis=1)

    oh_k = [(iota == sel_i[k]).astype(jnp.float32) for k in range(K)]
    cnt_tok = oh_k[0] + oh_k[1] + oh_k[2] + oh_k[3]
    ii = lax.broadcasted_iota(jnp.int32, (NB, NB), 0)
    jj = lax.broadcasted_iota(jnp.int32, (NB, NB), 1)
    tril = (jj < ii).astype(jnp.float32)
    c_excl = lax.dot_general(
        tril, cnt_tok, (((1,), (0,)), ((), ())),
        preferred_element_type=jnp.float32)
    base = c_excl + carry_ref[0:1, :]
    ranks = [jnp.sum(oh_k[k] * base, axis=1, keepdims=True) for k in range(K)]
    rank_ref[...] = jnp.concatenate(ranks, axis=1).astype(jnp.int32)
    carry_new = carry_ref[0:1, :] + jnp.sum(cnt_tok, axis=0, keepdims=True)
    carry_ref[0:1, :] = carry_new
    counts_ref[...] = carry_new.astype(jnp.int32)


def _routing(x2d, Wg, bg):
    n, d = x2d.shape
    grid = (n // NB,)
    return pl.pallas_call(
        _routing_body,
        grid=grid,
        in_specs=[
            pl.BlockSpec((NB, d), lambda i: (i, 0)),
            pl.BlockSpec((E, d), lambda i: (0, 0)),
            pl.BlockSpec((1, E), lambda i: (0, 0)),
        ],
        out_specs=[
            pl.BlockSpec((NB, E), lambda i: (i, 0)),
            pl.BlockSpec((NB, K), lambda i: (i, 0)),
            pl.BlockSpec((NB, K), lambda i: (i, 0)),
            pl.BlockSpec((NB, K), lambda i: (i, 0)),
            pl.BlockSpec((1, E), lambda i: (0, 0)),
        ],
        out_shape=[
            jax.ShapeDtypeStruct((n, E), jnp.float32),
            jax.ShapeDtypeStruct((n, K), jnp.float32),
            jax.ShapeDtypeStruct((n, K), jnp.int32),
            jax.ShapeDtypeStruct((n, K), jnp.int32),
            jax.ShapeDtypeStruct((1, E), jnp.int32),
        ],
        scratch_shapes=[pltpu.VMEM((8, E), jnp.float32)],
    )(x2d, Wg, bg.reshape(1, E))


def _dispatch_metadata(itop, wtop, rank, counts, n_tiles_max):
    nk = itop.shape[0] * K
    i32 = jnp.int32
    iflat = itop.reshape(-1).astype(i32)
    wflat = wtop.reshape(-1)
    rank = rank.reshape(-1)
    counts = counts.reshape(-1)
    p_tok = (jnp.arange(nk, dtype=i32) // K)
    tiles_e = (counts + T - 1) // T
    pad_sz = tiles_e * T
    pad_off = jnp.cumsum(pad_sz) - pad_sz
    slot = pad_off[iflat] + rank
    total_tiles = jnp.sum(tiles_e)
    tile_cum = jnp.cumsum(tiles_e)
    gidx = jnp.arange(n_tiles_max, dtype=i32)
    tile_expert = jnp.minimum(
        jnp.searchsorted(tile_cum, gidx, side="right").astype(i32), E - 1)
    tile_valid = (gidx < total_tiles).astype(i32)
    tile_blk = jnp.where(tile_valid == 1, gidx, total_tiles - 1).astype(i32)
    n_rows = n_tiles_max * T
    n_tok = itop.shape[0]
    tok_pad = (jnp.arange(n_rows, dtype=i32) % n_tok).at[slot].set(p_tok)
    w_pad = jnp.zeros((n_rows,), jnp.float32).at[slot].set(wflat)
    return slot, tok_pad, w_pad, tile_expert, tile_blk, tile_valid


def _sc_gather(src, idx, chunk):
    m = idx.shape[0]
    d = src.shape[1]
    per_w = m // _NW
    n_iter = per_w // chunk
    mesh = plsc.VectorSubcoreMesh(core_axis_name="c", subcore_axis_name="s")

    @functools.partial(
        pl.kernel,
        out_type=jax.ShapeDtypeStruct((m, d), src.dtype),
        mesh=mesh,
        scratch_types=[
            pltpu.VMEM((chunk,), jnp.int32),
            pltpu.VMEM((chunk, d), src.dtype),
            pltpu.SemaphoreType.DMA,
        ],
    )
    def gather_k(src_hbm, idx_hbm, out_hbm, idx_v, rows_v, sem):
        wid = lax.axis_index("s") * _NC + lax.axis_index("c")
        base = wid * per_w
        for i in range(n_iter):
            off = base + i * chunk
            pltpu.sync_copy(idx_hbm.at[pl.ds(off, chunk)], idx_v)
            pltpu.async_copy(src_hbm.at[idx_v], rows_v, sem).wait()
            pltpu.sync_copy(rows_v, out_hbm.at[pl.ds(off, chunk)])

    return gather_k(src, idx)


def _ffn_body(blk_ref, exp_ref, val_ref, xg_ref, w1_ref, b1_ref, w2_ref,
              b2_ref, ws_ref, out_ref):
    g = pl.program_id(0)

    @pl.when(val_ref[g] == 1)
    def _():
        x = xg_ref[...]
        h = lax.dot_general(
            x, w1_ref[0], (((1,), (1,)), ((), ())),
            preferred_element_type=jnp.float32)
        h = h + b1_ref[0]
        h = 0.5 * h * (1.0 + lax.erf(h * 0.7071067811865476))
        y = lax.dot_general(
            h, w2_ref[0], (((1,), (1,)), ((), ())),
            preferred_element_type=jnp.float32)
        y = y + b2_ref[0]
        out_ref[...] = y * ws_ref[...]


def _grouped_ffn(xg, W1, b1, W2, b2, w_pad, tile_blk, tile_expert, tile_valid,
                 n_tiles_max):
    d = W1.shape[2]
    h = W1.shape[1]
    n_rows = n_tiles_max * T
    grid_spec = pltpu.PrefetchScalarGridSpec(
        num_scalar_prefetch=3,
        grid=(n_tiles_max,),
        in_specs=[
            pl.BlockSpec((T, d), lambda g, blk, exp, val: (blk[g], 0)),
            pl.BlockSpec((1, h, d), lambda g, blk, exp, val: (exp[g], 0, 0)),
            pl.BlockSpec((1, 1, h), lambda g, blk, exp, val: (exp[g], 0, 0)),
            pl.BlockSpec((1, d, h), lambda g, blk, exp, val: (exp[g], 0, 0)),
            pl.BlockSpec((1, 1, d), lambda g, blk, exp, val: (exp[g], 0, 0)),
            pl.BlockSpec((T, 1), lambda g, blk, exp, val: (blk[g], 0)),
        ],
        out_specs=pl.BlockSpec((T, d), lambda g, blk, exp, val: (blk[g], 0)),
    )
    return pl.pallas_call(
        _ffn_body,
        grid_spec=grid_spec,
        out_shape=jax.ShapeDtypeStruct((n_rows, d), jnp.float32),
    )(tile_blk, tile_expert, tile_valid, xg, W1, b1.reshape(E, 1, h), W2,
      b2.reshape(E, 1, d), w_pad.reshape(n_rows, 1))


def _combine_body(yp_ref, out_ref):
    y = yp_ref[...]
    d = out_ref.shape[1]
    acc = y[:, 0:d] + y[:, d:2 * d]
    acc = acc + y[:, 2 * d:3 * d]
    out_ref[...] = acc + y[:, 3 * d:4 * d]


def _combine(yp, n, d):
    blk = 512
    y2 = yp.reshape(n, K * d)
    return pl.pallas_call(
        _combine_body,
        grid=(n // blk,),
        in_specs=[pl.BlockSpec((blk, K * d), lambda i: (i, 0))],
        out_specs=pl.BlockSpec((blk, d), lambda i: (i, 0)),
        out_shape=jax.ShapeDtypeStruct((n, d), jnp.float32),
    )(y2)


def kernel(x, Wg, bg, W1, b1, W2, b2):
    b, s, d = x.shape
    n = b * s
    nk = n * K
    n_tiles_max = nk // T + E
    x2d = x.reshape(n, d)

    logits, wtop, itop, rank, counts = _routing(x2d, Wg, bg)
    slot, tok_pad, w_pad, tile_expert, tile_blk, tile_valid = (
        _dispatch_metadata(itop, wtop, rank, counts, n_tiles_max))

    xg = _sc_gather(x2d, tok_pad, chunk=128)
    yg = _grouped_ffn(xg, W1, b1, W2, b2, w_pad, tile_blk, tile_expert,
                      tile_valid, n_tiles_max)
    yp = _sc_gather(yg, slot, chunk=128)
    final = _combine(yp, n, d)
    return final.reshape(b, s, d), logits


# speedup vs baseline: 6.2567x; 1.3165x over previous
import functools

import jax
import jax.numpy as jnp
from jax import lax
from jax.experimental import pallas as pl
from jax.experimental.pallas import tpu as pltpu
from jax.experimental.pallas import tpu_sc as plsc

E = 64
K = 4
T = 128
NB = 512

_NC, _NS = 2, 16
_NW = _NC * _NS


def _routing_body(x_ref, wg_ref, bg_ref, logits_ref, wtop_ref, itop_ref,
                  rank_ref, counts_ref, carry_ref):
    pid = pl.program_id(0)

    @pl.when(pid == 0)
    def _():
        carry_ref[...] = jnp.zeros_like(carry_ref)

    x = x_ref[...]
    wg = wg_ref[...]
    logits = lax.dot_general(
        x, wg, (((1,), (1,)), ((), ())),
        preferred_element_type=jnp.float32,
    ) + bg_ref[...]
    logits_ref[...] = logits

    iota = lax.broadcasted_iota(jnp.int32, logits.shape, 1)
    cur = logits
    sel_l, sel_i = [], []
    for _ in range(K):
        mk = jnp.max(cur, axis=1, keepdims=True)
        ik = jnp.min(jnp.where(cur == mk, iota, E), axis=1, keepdims=True)
        sel_l.append(mk)
        sel_i.append(ik)
        cur = jnp.where(iota == ik, -1e30, cur)
    m0 = sel_l[0]
    exps = [jnp.exp(l - m0) for l in sel_l]
    denom = exps[0] + exps[1] + exps[2] + exps[3]
    wtop_ref[...] = jnp.concatenate([e / denom for e in exps], axis=1)
    itop_ref[...] = jnp.concatenate(sel_i, axis=1)

    oh_k = [(iota == sel_i[k]).astype(jnp.float32) for k in range(K)]
    cnt_tok = oh_k[0] + oh_k[1] + oh_k[2] + oh_k[3]
    ii = lax.broadcasted_iota(jnp.int32, (NB, NB), 0)
    jj = lax.broadcasted_iota(jnp.int32, (NB, NB), 1)
    tril = (jj < ii).astype(jnp.float32)
    c_excl = lax.dot_general(
        tril, cnt_tok, (((1,), (0,)), ((), ())),
        preferred_element_type=jnp.float32)
    base = c_excl + carry_ref[0:1, :]
    ranks = [jnp.sum(oh_k[k] * base, axis=1, keepdims=True) for k in range(K)]
    rank_ref[...] = jnp.concatenate(ranks, axis=1).astype(jnp.int32)
    carry_new = carry_ref[0:1, :] + jnp.sum(cnt_tok, axis=0, keepdims=True)
    carry_ref[0:1, :] = carry_new
    counts_ref[...] = carry_new.astype(jnp.int32)


def _routing(x2d, Wg, bg):
    n, d = x2d.shape
    grid = (n // NB,)
    return pl.pallas_call(
        _routing_body,
        grid=grid,
        in_specs=[
            pl.BlockSpec((NB, d), lambda i: (i, 0)),
            pl.BlockSpec((E, d), lambda i: (0, 0)),
            pl.BlockSpec((1, E), lambda i: (0, 0)),
        ],
        out_specs=[
            pl.BlockSpec((NB, E), lambda i: (i, 0)),
            pl.BlockSpec((NB, K), lambda i: (i, 0)),
            pl.BlockSpec((NB, K), lambda i: (i, 0)),
            pl.BlockSpec((NB, K), lambda i: (i, 0)),
            pl.BlockSpec((1, E), lambda i: (0, 0)),
        ],
        out_shape=[
            jax.ShapeDtypeStruct((n, E), jnp.float32),
            jax.ShapeDtypeStruct((n, K), jnp.float32),
            jax.ShapeDtypeStruct((n, K), jnp.int32),
            jax.ShapeDtypeStruct((n, K), jnp.int32),
            jax.ShapeDtypeStruct((1, E), jnp.int32),
        ],
        scratch_shapes=[pltpu.VMEM((8, E), jnp.float32)],
    )(x2d, Wg, bg.reshape(1, E))


def _dispatch_metadata(itop, rank, counts, n_tiles_max):
    i32 = jnp.int32
    iflat = itop.reshape(-1).astype(i32)
    rank = rank.reshape(-1)
    counts = counts.reshape(-1)
    tiles_e = (counts + T - 1) // T
    pad_sz = tiles_e * T
    pad_off = jnp.cumsum(pad_sz) - pad_sz
    slot = pad_off[iflat] + rank
    total_tiles = jnp.sum(tiles_e)
    tile_cum = jnp.cumsum(tiles_e)
    gidx = jnp.arange(n_tiles_max, dtype=i32)
    tile_expert = jnp.minimum(
        jnp.searchsorted(tile_cum, gidx, side="right").astype(i32), E - 1)
    tile_valid = (gidx < total_tiles).astype(i32)
    tile_blk = jnp.where(tile_valid == 1, gidx, total_tiles - 1).astype(i32)
    n = itop.shape[0]
    slot_km = slot.reshape(n, K).T.reshape(-1)
    return slot_km, tile_expert, tile_blk, tile_valid


def _sc_scatter_dispatch(src, idx, n_rows_out, chunk):
    m = idx.shape[0]
    n, d = src.shape
    per_w = m // _NW
    n_iter = per_w // chunk
    planes_per_w = n // per_w
    mesh = plsc.VectorSubcoreMesh(core_axis_name="c", subcore_axis_name="s")

    @functools.partial(
        pl.kernel,
        out_type=jax.ShapeDtypeStruct((n_rows_out, d), src.dtype),
        mesh=mesh,
        scratch_types=[
            pltpu.VMEM((chunk,), jnp.int32),
            pltpu.VMEM((chunk, d), src.dtype),
            pltpu.SemaphoreType.DMA,
        ],
    )
    def scatter_k(src_hbm, idx_hbm, out_hbm, idx_v, rows_v, sem):
        wid = lax.axis_index("s") * _NC + lax.axis_index("c")
        base = wid * per_w
        t_base = (wid % planes_per_w) * per_w
        for i in range(n_iter):
            pltpu.sync_copy(idx_hbm.at[pl.ds(base + i * chunk, chunk)], idx_v)
            pltpu.sync_copy(src_hbm.at[pl.ds(t_base + i * chunk, chunk)], rows_v)
            pltpu.async_copy(rows_v, out_hbm.at[idx_v], sem).wait()

    return scatter_k(src, idx)


def _sc_gather(src, idx, chunk):
    m = idx.shape[0]
    d = src.shape[1]
    per_w = m // _NW
    n_iter = per_w // chunk
    mesh = plsc.VectorSubcoreMesh(core_axis_name="c", subcore_axis_name="s")

    @functools.partial(
        pl.kernel,
        out_type=jax.ShapeDtypeStruct((m, d), src.dtype),
        mesh=mesh,
        scratch_types=[
            pltpu.VMEM((chunk,), jnp.int32),
            pltpu.VMEM((chunk, d), src.dtype),
            pltpu.SemaphoreType.DMA,
        ],
    )
    def gather_k(src_hbm, idx_hbm, out_hbm, idx_v, rows_v, sem):
        wid = lax.axis_index("s") * _NC + lax.axis_index("c")
        base = wid * per_w
        for i in range(n_iter):
            off = base + i * chunk
            pltpu.sync_copy(idx_hbm.at[pl.ds(off, chunk)], idx_v)
            pltpu.async_copy(src_hbm.at[idx_v], rows_v, sem).wait()
            pltpu.sync_copy(rows_v, out_hbm.at[pl.ds(off, chunk)])

    return gather_k(src, idx)


def _ffn_body(blk_ref, exp_ref, val_ref, xg_ref, w1_ref, b1_ref, w2_ref,
              b2_ref, out_ref):
    g = pl.program_id(0)

    @pl.when(val_ref[g] == 1)
    def _():
        x = xg_ref[...]
        h = lax.dot_general(
            x, w1_ref[0], (((1,), (1,)), ((), ())),
            preferred_element_type=jnp.float32)
        h = h + b1_ref[0]
        h = 0.5 * h * (1.0 + lax.erf(h * 0.7071067811865476))
        y = lax.dot_general(
            h, w2_ref[0], (((1,), (1,)), ((), ())),
            preferred_element_type=jnp.float32)
        out_ref[...] = y + b2_ref[0]


def _grouped_ffn(xg, W1, b1, W2, b2, tile_blk, tile_expert, tile_valid,
                 n_tiles_max):
    d = W1.shape[2]
    h = W1.shape[1]
    n_rows = n_tiles_max * T
    grid_spec = pltpu.PrefetchScalarGridSpec(
        num_scalar_prefetch=3,
        grid=(n_tiles_max,),
        in_specs=[
            pl.BlockSpec((T, d), lambda g, blk, exp, val: (blk[g], 0)),
            pl.BlockSpec((1, h, d), lambda g, blk, exp, val: (exp[g], 0, 0)),
            pl.BlockSpec((1, 1, h), lambda g, blk, exp, val: (exp[g], 0, 0)),
            pl.BlockSpec((1, d, h), lambda g, blk, exp, val: (exp[g], 0, 0)),
            pl.BlockSpec((1, 1, d), lambda g, blk, exp, val: (exp[g], 0, 0)),
        ],
        out_specs=pl.BlockSpec((T, d), lambda g, blk, exp, val: (blk[g], 0)),
    )
    return pl.pallas_call(
        _ffn_body,
        grid_spec=grid_spec,
        out_shape=jax.ShapeDtypeStruct((n_rows, d), jnp.float32),
    )(tile_blk, tile_expert, tile_valid, xg, W1, b1.reshape(E, 1, h), W2,
      b2.reshape(E, 1, d))


def _combine_body(y0_ref, y1_ref, y2_ref, y3_ref, w_ref, out_ref):
    w = w_ref[...]
    acc = y0_ref[...] * w[:, 0:1]
    acc = acc + y1_ref[...] * w[:, 1:2]
    acc = acc + y2_ref[...] * w[:, 2:3]
    out_ref[...] = acc + y3_ref[...] * w[:, 3:4]


def _combine(yp, wtop, n, d):
    blk = 512
    nb = n // blk
    plane_specs = [
        pl.BlockSpec((blk, d), (lambda i, kk=kk: (i + kk * nb, 0)))
        for kk in range(K)
    ]
    return pl.pallas_call(
        _combine_body,
        grid=(nb,),
        in_specs=plane_specs + [pl.BlockSpec((blk, K), lambda i: (i, 0))],
        out_specs=pl.BlockSpec((blk, d), lambda i: (i, 0)),
        out_shape=jax.ShapeDtypeStruct((n, d), jnp.float32),
    )(yp, yp, yp, yp, wtop)


def kernel(x, Wg, bg, W1, b1, W2, b2):
    b, s, d = x.shape
    n = b * s
    nk = n * K
    n_tiles_max = nk // T + E
    x2d = x.reshape(n, d)

    logits, wtop, itop, rank, counts = _routing(x2d, Wg, bg)
    slot_km, tile_expert, tile_blk, tile_valid = (
        _dispatch_metadata(itop, rank, counts, n_tiles_max))

    n_rows = n_tiles_max * T
    xg = _sc_scatter_dispatch(x2d, slot_km, n_rows, chunk=128)
    yg = _grouped_ffn(xg, W1, b1, W2, b2, tile_blk, tile_expert,
                      tile_valid, n_tiles_max)
    yp = _sc_gather(yg, slot_km, chunk=128)
    final = _combine(yp, wtop, n, d)
    return final.reshape(b, s, d), logits


# speedup vs baseline: 7.7106x; 1.2324x over previous
import functools

import jax
import jax.numpy as jnp
from jax import lax
from jax.experimental import pallas as pl
from jax.experimental.pallas import tpu as pltpu
from jax.experimental.pallas import tpu_sc as plsc

E = 64
K = 4
T = 256
NB = 512

_NC, _NS = 2, 16
_NW = _NC * _NS


def _routing_body(x_ref, wg_ref, bg_ref, logits_ref, wtop_ref, itop_ref,
                  rank_ref, counts_ref, carry_ref):
    pid = pl.program_id(0)

    @pl.when(pid == 0)
    def _():
        carry_ref[...] = jnp.zeros_like(carry_ref)

    x = x_ref[...]
    wg = wg_ref[...]
    logits = lax.dot_general(
        x, wg, (((1,), (1,)), ((), ())),
        preferred_element_type=jnp.float32,
    ) + bg_ref[...]
    logits_ref[...] = logits

    iota = lax.broadcasted_iota(jnp.int32, logits.shape, 1)
    cur = logits
    sel_l, sel_i = [], []
    for _ in range(K):
        mk = jnp.max(cur, axis=1, keepdims=True)
        ik = jnp.min(jnp.where(cur == mk, iota, E), axis=1, keepdims=True)
        sel_l.append(mk)
        sel_i.append(ik)
        cur = jnp.where(iota == ik, -1e30, cur)
    m0 = sel_l[0]
    exps = [jnp.exp(l - m0) for l in sel_l]
    denom = exps[0] + exps[1] + exps[2] + exps[3]
    wtop_ref[...] = jnp.concatenate([e / denom for e in exps], axis=1)
    itop_ref[...] = jnp.concatenate(sel_i, axis=1)

    oh_k = [(iota == sel_i[k]).astype(jnp.float32) for k in range(K)]
    cnt_tok = oh_k[0] + oh_k[1] + oh_k[2] + oh_k[3]
    ii = lax.broadcasted_iota(jnp.int32, (NB, NB), 0)
    jj = lax.broadcasted_iota(jnp.int32, (NB, NB), 1)
    tril = (jj < ii).astype(jnp.float32)
    c_excl = lax.dot_general(
        tril, cnt_tok, (((1,), (0,)), ((), ())),
        preferred_element_type=jnp.float32)
    base = c_excl + carry_ref[0:1, :]
    ranks = [jnp.sum(oh_k[k] * base, axis=1, keepdims=True) for k in range(K)]
    rank_ref[...] = jnp.concatenate(ranks, axis=1).astype(jnp.int32)
    carry_new = carry_ref[0:1, :] + jnp.sum(cnt_tok, axis=0, keepdims=True)
    carry_ref[0:1, :] = carry_new
    counts_ref[...] = carry_new.astype(jnp.int32)


def _routing(x2d, Wg, bg):
    n, d = x2d.shape
    grid = (n // NB,)
    return pl.pallas_call(
        _routing_body,
        grid=grid,
        in_specs=[
            pl.BlockSpec((NB, d), lambda i: (i, 0)),
            pl.BlockSpec((E, d), lambda i: (0, 0)),
            pl.BlockSpec((1, E), lambda i: (0, 0)),
        ],
        out_specs=[
            pl.BlockSpec((NB, E), lambda i: (i, 0)),
            pl.BlockSpec((NB, K), lambda i: (i, 0)),
            pl.BlockSpec((NB, K), lambda i: (i, 0)),
            pl.BlockSpec((NB, K), lambda i: (i, 0)),
            pl.BlockSpec((1, E), lambda i: (0, 0)),
        ],
        out_shape=[
            jax.ShapeDtypeStruct((n, E), jnp.float32),
            jax.ShapeDtypeStruct((n, K), jnp.float32),
            jax.ShapeDtypeStruct((n, K), jnp.int32),
            jax.ShapeDtypeStruct((n, K), jnp.int32),
            jax.ShapeDtypeStruct((1, E), jnp.int32),
        ],
        scratch_shapes=[pltpu.VMEM((8, E), jnp.float32)],
    )(x2d, Wg, bg.reshape(1, E))


def _dispatch_metadata(itop, rank, counts, n_tiles_max):
    i32 = jnp.int32
    iflat = itop.reshape(-1).astype(i32)
    rank = rank.reshape(-1)
    counts = counts.reshape(-1)
    tiles_e = (counts + T - 1) // T
    pad_sz = tiles_e * T
    pad_off = jnp.cumsum(pad_sz) - pad_sz
    slot = pad_off[iflat] + rank
    total_tiles = jnp.sum(tiles_e)
    tile_cum = jnp.cumsum(tiles_e)
    gidx = jnp.arange(n_tiles_max, dtype=i32)
    tile_expert = jnp.minimum(
        jnp.searchsorted(tile_cum, gidx, side="right").astype(i32), E - 1)
    tile_valid = (gidx < total_tiles).astype(i32)
    tile_blk = jnp.where(tile_valid == 1, gidx, total_tiles - 1).astype(i32)
    n = itop.shape[0]
    slot_km = slot.reshape(n, K).T.reshape(-1)
    return slot_km, tile_expert, tile_blk, tile_valid


def _sc_scatter_dispatch(src, idx, n_rows_out, chunk):
    m = idx.shape[0]
    n, d = src.shape
    per_w = m // _NW
    n_iter = per_w // chunk
    planes_per_w = n // per_w
    mesh = plsc.VectorSubcoreMesh(core_axis_name="c", subcore_axis_name="s")

    @functools.partial(
        pl.kernel,
        out_type=jax.ShapeDtypeStruct((n_rows_out, d), src.dtype),
        mesh=mesh,
        scratch_types=[
            pltpu.VMEM((chunk,), jnp.int32),
            pltpu.VMEM((chunk, d), src.dtype),
            pltpu.SemaphoreType.DMA,
        ],
    )
    def scatter_k(src_hbm, idx_hbm, out_hbm, idx_v, rows_v, sem):
        wid = lax.axis_index("s") * _NC + lax.axis_index("c")
        base = wid * per_w
        t_base = (wid % planes_per_w) * per_w
        for i in range(n_iter):
            pltpu.sync_copy(idx_hbm.at[pl.ds(base + i * chunk, chunk)], idx_v)
            pltpu.sync_copy(src_hbm.at[pl.ds(t_base + i * chunk, chunk)], rows_v)
            pltpu.async_copy(rows_v, out_hbm.at[idx_v], sem).wait()

    return scatter_k(src, idx)


def _sc_gather(src, idx, chunk):
    m = idx.shape[0]
    d = src.shape[1]
    per_w = m // _NW
    n_iter = per_w // chunk
    mesh = plsc.VectorSubcoreMesh(core_axis_name="c", subcore_axis_name="s")

    @functools.partial(
        pl.kernel,
        out_type=jax.ShapeDtypeStruct((m, d), src.dtype),
        mesh=mesh,
        scratch_types=[
            pltpu.VMEM((chunk,), jnp.int32),
            pltpu.VMEM((chunk, d), src.dtype),
            pltpu.SemaphoreType.DMA,
        ],
    )
    def gather_k(src_hbm, idx_hbm, out_hbm, idx_v, rows_v, sem):
        wid = lax.axis_index("s") * _NC + lax.axis_index("c")
        base = wid * per_w
        for i in range(n_iter):
            off = base + i * chunk
            pltpu.sync_copy(idx_hbm.at[pl.ds(off, chunk)], idx_v)
            pltpu.async_copy(src_hbm.at[idx_v], rows_v, sem).wait()
            pltpu.sync_copy(rows_v, out_hbm.at[pl.ds(off, chunk)])

    return gather_k(src, idx)


def _ffn_body(blk_ref, exp_ref, val_ref, xg_ref, w1_ref, b1_ref, w2_ref,
              b2_ref, out_ref):
    g = pl.program_id(0)

    @pl.when(val_ref[g] == 1)
    def _():
        x = xg_ref[...]
        h = lax.dot_general(
            x, w1_ref[0], (((1,), (1,)), ((), ())),
            preferred_element_type=jnp.float32)
        h = h + b1_ref[0]
        h = 0.5 * h * (1.0 + lax.erf(h * 0.7071067811865476))
        y = lax.dot_general(
            h, w2_ref[0], (((1,), (1,)), ((), ())),
            preferred_element_type=jnp.float32)
        out_ref[...] = y + b2_ref[0]


def _grouped_ffn(xg, W1, b1, W2, b2, tile_blk, tile_expert, tile_valid,
                 n_tiles_max):
    d = W1.shape[2]
    h = W1.shape[1]
    n_rows = n_tiles_max * T
    grid_spec = pltpu.PrefetchScalarGridSpec(
        num_scalar_prefetch=3,
        grid=(n_tiles_max,),
        in_specs=[
            pl.BlockSpec((T, d), lambda g, blk, exp, val: (blk[g], 0)),
            pl.BlockSpec((1, h, d), lambda g, blk, exp, val: (exp[g], 0, 0)),
            pl.BlockSpec((1, 1, h), lambda g, blk, exp, val: (exp[g], 0, 0)),
            pl.BlockSpec((1, d, h), lambda g, blk, exp, val: (exp[g], 0, 0)),
            pl.BlockSpec((1, 1, d), lambda g, blk, exp, val: (exp[g], 0, 0)),
        ],
        out_specs=pl.BlockSpec((T, d), lambda g, blk, exp, val: (blk[g], 0)),
    )
    return pl.pallas_call(
        _ffn_body,
        grid_spec=grid_spec,
        out_shape=jax.ShapeDtypeStruct((n_rows, d), jnp.float32),
    )(tile_blk, tile_expert, tile_valid, xg, W1, b1.reshape(E, 1, h), W2,
      b2.reshape(E, 1, d))


def _combine_body(y0_ref, y1_ref, y2_ref, y3_ref, w_ref, out_ref):
    w = w_ref[...]
    acc = y0_ref[...] * w[:, 0:1]
    acc = acc + y1_ref[...] * w[:, 1:2]
    acc = acc + y2_ref[...] * w[:, 2:3]
    out_ref[...] = acc + y3_ref[...] * w[:, 3:4]


def _combine(yp, wtop, n, d):
    blk = 512
    nb = n // blk
    plane_specs = [
        pl.BlockSpec((blk, d), (lambda i, kk=kk: (i + kk * nb, 0)))
        for kk in range(K)
    ]
    return pl.pallas_call(
        _combine_body,
        grid=(nb,),
        in_specs=plane_specs + [pl.BlockSpec((blk, K), lambda i: (i, 0))],
        out_specs=pl.BlockSpec((blk, d), lambda i: (i, 0)),
        out_shape=jax.ShapeDtypeStruct((n, d), jnp.float32),
    )(yp, yp, yp, yp, wtop)


def kernel(x, Wg, bg, W1, b1, W2, b2):
    b, s, d = x.shape
    n = b * s
    nk = n * K
    n_tiles_max = nk // T + E
    x2d = x.reshape(n, d)

    logits, wtop, itop, rank, counts = _routing(x2d, Wg, bg)
    slot_km, tile_expert, tile_blk, tile_valid = (
        _dispatch_metadata(itop, rank, counts, n_tiles_max))

    n_rows = n_tiles_max * T
    xg = _sc_scatter_dispatch(x2d, slot_km, n_rows, chunk=128)
    yg = _grouped_ffn(xg, W1, b1, W2, b2, tile_blk, tile_expert,
                      tile_valid, n_tiles_max)
    yp = _sc_gather(yg, slot_km, chunk=128)
    final = _combine(yp, wtop, n, d)
    return final.reshape(b, s, d), logits


# speedup vs baseline: 8.2965x; 1.0760x over previous
import functools

import jax
import jax.numpy as jnp
from jax import lax
from jax.experimental import pallas as pl
from jax.experimental.pallas import tpu as pltpu
from jax.experimental.pallas import tpu_sc as plsc

E = 64
K = 4
T = 288
NB = 512

_NC, _NS = 2, 16
_NW = _NC * _NS


def _routing_body(x_ref, wg_ref, bg_ref, logits_ref, wtop_ref, itop_ref,
                  rank_ref, counts_ref, carry_ref):
    pid = pl.program_id(0)

    @pl.when(pid == 0)
    def _():
        carry_ref[...] = jnp.zeros_like(carry_ref)

    x = x_ref[...]
    wg = wg_ref[...]
    logits = lax.dot_general(
        x, wg, (((1,), (1,)), ((), ())),
        preferred_element_type=jnp.float32,
    ) + bg_ref[...]
    logits_ref[...] = logits

    iota = lax.broadcasted_iota(jnp.int32, logits.shape, 1)
    cur = logits
    sel_l, sel_i = [], []
    for _ in range(K):
        mk = jnp.max(cur, axis=1, keepdims=True)
        ik = jnp.min(jnp.where(cur == mk, iota, E), axis=1, keepdims=True)
        sel_l.append(mk)
        sel_i.append(ik)
        cur = jnp.where(iota == ik, -1e30, cur)
    m0 = sel_l[0]
    exps = [jnp.exp(l - m0) for l in sel_l]
    denom = exps[0] + exps[1] + exps[2] + exps[3]
    wtop_ref[...] = jnp.concatenate([e / denom for e in exps], axis=1)
    itop_ref[...] = jnp.concatenate(sel_i, axis=1)

    oh_k = [(iota == sel_i[k]).astype(jnp.float32) for k in range(K)]
    cnt_tok = oh_k[0] + oh_k[1] + oh_k[2] + oh_k[3]
    ii = lax.broadcasted_iota(jnp.int32, (NB, NB), 0)
    jj = lax.broadcasted_iota(jnp.int32, (NB, NB), 1)
    tril = (jj < ii).astype(jnp.float32)
    c_excl = lax.dot_general(
        tril, cnt_tok, (((1,), (0,)), ((), ())),
        preferred_element_type=jnp.float32)
    base = c_excl + carry_ref[0:1, :]
    ranks = [jnp.sum(oh_k[k] * base, axis=1, keepdims=True) for k in range(K)]
    rank_ref[...] = jnp.concatenate(ranks, axis=1).astype(jnp.int32)
    carry_new = carry_ref[0:1, :] + jnp.sum(cnt_tok, axis=0, keepdims=True)
    carry_ref[0:1, :] = carry_new
    counts_ref[...] = carry_new.astype(jnp.int32)


def _routing(x2d, Wg, bg):
    n, d = x2d.shape
    grid = (n // NB,)
    return pl.pallas_call(
        _routing_body,
        grid=grid,
        in_specs=[
            pl.BlockSpec((NB, d), lambda i: (i, 0)),
            pl.BlockSpec((E, d), lambda i: (0, 0)),
            pl.BlockSpec((1, E), lambda i: (0, 0)),
        ],
        out_specs=[
            pl.BlockSpec((NB, E), lambda i: (i, 0)),
            pl.BlockSpec((NB, K), lambda i: (i, 0)),
            pl.BlockSpec((NB, K), lambda i: (i, 0)),
            pl.BlockSpec((NB, K), lambda i: (i, 0)),
            pl.BlockSpec((1, E), lambda i: (0, 0)),
        ],
        out_shape=[
            jax.ShapeDtypeStruct((n, E), jnp.float32),
            jax.ShapeDtypeStruct((n, K), jnp.float32),
            jax.ShapeDtypeStruct((n, K), jnp.int32),
            jax.ShapeDtypeStruct((n, K), jnp.int32),
            jax.ShapeDtypeStruct((1, E), jnp.int32),
        ],
        scratch_shapes=[pltpu.VMEM((8, E), jnp.float32)],
    )(x2d, Wg, bg.reshape(1, E))


def _dispatch_metadata(itop, rank, counts, n_tiles_max):
    i32 = jnp.int32
    iflat = itop.reshape(-1).astype(i32)
    rank = rank.reshape(-1)
    counts = counts.reshape(-1)
    tiles_e = (counts + T - 1) // T
    pad_sz = tiles_e * T
    pad_off = jnp.cumsum(pad_sz) - pad_sz
    slot = pad_off[iflat] + rank
    total_tiles = jnp.sum(tiles_e)
    tile_cum = jnp.cumsum(tiles_e)
    gidx = jnp.arange(n_tiles_max, dtype=i32)
    tile_expert = jnp.minimum(
        jnp.searchsorted(tile_cum, gidx, side="right").astype(i32), E - 1)
    tile_valid = (gidx < total_tiles).astype(i32)
    tile_blk = jnp.where(tile_valid == 1, gidx, total_tiles - 1).astype(i32)
    n = itop.shape[0]
    slot_km = slot.reshape(n, K).T.reshape(-1)
    return slot_km, tile_expert, tile_blk, tile_valid


def _sc_scatter_dispatch(src, idx, n_rows_out, chunk):
    m = idx.shape[0]
    n, d = src.shape
    per_w = m // _NW
    n_iter = per_w // chunk
    planes_per_w = n // per_w
    mesh = plsc.VectorSubcoreMesh(core_axis_name="c", subcore_axis_name="s")

    @functools.partial(
        pl.kernel,
        out_type=jax.ShapeDtypeStruct((n_rows_out, d), src.dtype),
        mesh=mesh,
        scratch_types=[
            pltpu.VMEM((chunk,), jnp.int32),
            pltpu.VMEM((chunk, d), src.dtype),
            pltpu.SemaphoreType.DMA,
        ],
    )
    def scatter_k(src_hbm, idx_hbm, out_hbm, idx_v, rows_v, sem):
        wid = lax.axis_index("s") * _NC + lax.axis_index("c")
        base = wid * per_w
        t_base = (wid % planes_per_w) * per_w
        for i in range(n_iter):
            pltpu.sync_copy(idx_hbm.at[pl.ds(base + i * chunk, chunk)], idx_v)
            pltpu.sync_copy(src_hbm.at[pl.ds(t_base + i * chunk, chunk)], rows_v)
            pltpu.async_copy(rows_v, out_hbm.at[idx_v], sem).wait()

    return scatter_k(src, idx)


def _sc_gather(src, idx, chunk):
    m = idx.shape[0]
    d = src.shape[1]
    per_w = m // _NW
    n_iter = per_w // chunk
    mesh = plsc.VectorSubcoreMesh(core_axis_name="c", subcore_axis_name="s")

    @functools.partial(
        pl.kernel,
        out_type=jax.ShapeDtypeStruct((m, d), src.dtype),
        mesh=mesh,
        scratch_types=[
            pltpu.VMEM((chunk,), jnp.int32),
            pltpu.VMEM((chunk, d), src.dtype),
            pltpu.SemaphoreType.DMA,
        ],
    )
    def gather_k(src_hbm, idx_hbm, out_hbm, idx_v, rows_v, sem):
        wid = lax.axis_index("s") * _NC + lax.axis_index("c")
        base = wid * per_w
        for i in range(n_iter):
            off = base + i * chunk
            pltpu.sync_copy(idx_hbm.at[pl.ds(off, chunk)], idx_v)
            pltpu.async_copy(src_hbm.at[idx_v], rows_v, sem).wait()
            pltpu.sync_copy(rows_v, out_hbm.at[pl.ds(off, chunk)])

    return gather_k(src, idx)


def _ffn_body(blk_ref, exp_ref, val_ref, xg_ref, w1_ref, b1_ref, w2_ref,
              b2_ref, out_ref):
    g = pl.program_id(0)

    @pl.when(val_ref[g] == 1)
    def _():
        x = xg_ref[...]
        h = lax.dot_general(
            x, w1_ref[0], (((1,), (1,)), ((), ())),
            preferred_element_type=jnp.float32)
        h = h + b1_ref[0]
        h = 0.5 * h * (1.0 + lax.erf(h * 0.7071067811865476))
        y = lax.dot_general(
            h, w2_ref[0], (((1,), (1,)), ((), ())),
            preferred_element_type=jnp.float32)
        out_ref[...] = y + b2_ref[0]


def _grouped_ffn(xg, W1, b1, W2, b2, tile_blk, tile_expert, tile_valid,
                 n_tiles_max):
    d = W1.shape[2]
    h = W1.shape[1]
    n_rows = n_tiles_max * T
    grid_spec = pltpu.PrefetchScalarGridSpec(
        num_scalar_prefetch=3,
        grid=(n_tiles_max,),
        in_specs=[
            pl.BlockSpec((T, d), lambda g, blk, exp, val: (blk[g], 0)),
            pl.BlockSpec((1, h, d), lambda g, blk, exp, val: (exp[g], 0, 0)),
            pl.BlockSpec((1, 1, h), lambda g, blk, exp, val: (exp[g], 0, 0)),
            pl.BlockSpec((1, d, h), lambda g, blk, exp, val: (exp[g], 0, 0)),
            pl.BlockSpec((1, 1, d), lambda g, blk, exp, val: (exp[g], 0, 0)),
        ],
        out_specs=pl.BlockSpec((T, d), lambda g, blk, exp, val: (blk[g], 0)),
    )
    return pl.pallas_call(
        _ffn_body,
        grid_spec=grid_spec,
        out_shape=jax.ShapeDtypeStruct((n_rows, d), jnp.float32),
    )(tile_blk, tile_expert, tile_valid, xg, W1, b1.reshape(E, 1, h), W2,
      b2.reshape(E, 1, d))


def _combine_body(y0_ref, y1_ref, y2_ref, y3_ref, w_ref, out_ref):
    w = w_ref[...]
    acc = y0_ref[...] * w[:, 0:1]
    acc = acc + y1_ref[...] * w[:, 1:2]
    acc = acc + y2_ref[...] * w[:, 2:3]
    out_ref[...] = acc + y3_ref[...] * w[:, 3:4]


def _combine(yp, wtop, n, d):
    blk = 512
    nb = n // blk
    plane_specs = [
        pl.BlockSpec((blk, d), (lambda i, kk=kk: (i + kk * nb, 0)))
        for kk in range(K)
    ]
    return pl.pallas_call(
        _combine_body,
        grid=(nb,),
        in_specs=plane_specs + [pl.BlockSpec((blk, K), lambda i: (i, 0))],
        out_specs=pl.BlockSpec((blk, d), lambda i: (i, 0)),
        out_shape=jax.ShapeDtypeStruct((n, d), jnp.float32),
    )(yp, yp, yp, yp, wtop)


def kernel(x, Wg, bg, W1, b1, W2, b2):
    b, s, d = x.shape
    n = b * s
    nk = n * K
    n_tiles_max = nk // T + E
    x2d = x.reshape(n, d)

    logits, wtop, itop, rank, counts = _routing(x2d, Wg, bg)
    slot_km, tile_expert, tile_blk, tile_valid = (
        _dispatch_metadata(itop, rank, counts, n_tiles_max))

    n_rows = n_tiles_max * T
    xg = _sc_scatter_dispatch(x2d, slot_km, n_rows, chunk=128)
    yg = _grouped_ffn(xg, W1, b1, W2, b2, tile_blk, tile_expert,
                      tile_valid, n_tiles_max)
    yp = _sc_gather(yg, slot_km, chunk=128)
    final = _combine(yp, wtop, n, d)
    return final.reshape(b, s, d), logits


# speedup vs baseline: 9.7355x; 1.1734x over previous
import functools

import jax
import jax.numpy as jnp
from jax import lax
from jax.experimental import pallas as pl
from jax.experimental.pallas import tpu as pltpu
from jax.experimental.pallas import tpu_sc as plsc

E = 64
K = 4
T = 288
NB = 512

_NC, _NS = 2, 16
_NW = _NC * _NS


def _routing_body(x_ref, wg_ref, bg_ref, logits_ref, wtop_ref, itop_ref,
                  rank_ref, counts_ref, carry_ref):
    pid = pl.program_id(0)

    @pl.when(pid == 0)
    def _():
        carry_ref[...] = jnp.zeros_like(carry_ref)

    x = x_ref[...]
    wg = wg_ref[...]
    logits = lax.dot_general(
        x, wg, (((1,), (1,)), ((), ())),
        preferred_element_type=jnp.float32,
    ) + bg_ref[...]
    logits_ref[...] = logits

    iota = lax.broadcasted_iota(jnp.int32, logits.shape, 1)
    cur = logits
    sel_l, sel_i = [], []
    for _ in range(K):
        mk = jnp.max(cur, axis=1, keepdims=True)
        ik = jnp.min(jnp.where(cur == mk, iota, E), axis=1, keepdims=True)
        sel_l.append(mk)
        sel_i.append(ik)
        cur = jnp.where(iota == ik, -1e30, cur)
    m0 = sel_l[0]
    exps = [jnp.exp(l - m0) for l in sel_l]
    denom = exps[0] + exps[1] + exps[2] + exps[3]
    wtop_ref[...] = jnp.concatenate([e / denom for e in exps], axis=1)
    itop_ref[...] = jnp.concatenate(sel_i, axis=1)

    oh_k = [(iota == sel_i[k]).astype(jnp.float32) for k in range(K)]
    cnt_tok = oh_k[0] + oh_k[1] + oh_k[2] + oh_k[3]
    ii = lax.broadcasted_iota(jnp.int32, (NB, NB), 0)
    jj = lax.broadcasted_iota(jnp.int32, (NB, NB), 1)
    tril = (jj < ii).astype(jnp.float32)
    c_excl = lax.dot_general(
        tril, cnt_tok, (((1,), (0,)), ((), ())),
        preferred_element_type=jnp.float32)
    base = c_excl + carry_ref[0:1, :]
    ranks = [jnp.sum(oh_k[k] * base, axis=1, keepdims=True) for k in range(K)]
    rank_ref[...] = jnp.concatenate(ranks, axis=1).astype(jnp.int32)
    carry_new = carry_ref[0:1, :] + jnp.sum(cnt_tok, axis=0, keepdims=True)
    carry_ref[0:1, :] = carry_new
    counts_ref[...] = carry_new.astype(jnp.int32)


def _routing(x2d, Wg, bg):
    n, d = x2d.shape
    grid = (n // NB,)
    return pl.pallas_call(
        _routing_body,
        grid=grid,
        in_specs=[
            pl.BlockSpec((NB, d), lambda i: (i, 0)),
            pl.BlockSpec((E, d), lambda i: (0, 0)),
            pl.BlockSpec((1, E), lambda i: (0, 0)),
        ],
        out_specs=[
            pl.BlockSpec((NB, E), lambda i: (i, 0)),
            pl.BlockSpec((NB, K), lambda i: (i, 0)),
            pl.BlockSpec((NB, K), lambda i: (i, 0)),
            pl.BlockSpec((NB, K), lambda i: (i, 0)),
            pl.BlockSpec((1, E), lambda i: (0, 0)),
        ],
        out_shape=[
            jax.ShapeDtypeStruct((n, E), jnp.float32),
            jax.ShapeDtypeStruct((n, K), jnp.float32),
            jax.ShapeDtypeStruct((n, K), jnp.int32),
            jax.ShapeDtypeStruct((n, K), jnp.int32),
            jax.ShapeDtypeStruct((1, E), jnp.int32),
        ],
        scratch_shapes=[pltpu.VMEM((8, E), jnp.float32)],
    )(x2d, Wg, bg.reshape(1, E))


def _dispatch_metadata(itop, rank, counts, n_tiles_max):
    i32 = jnp.int32
    iflat = itop.reshape(-1).astype(i32)
    rank = rank.reshape(-1)
    counts = counts.reshape(-1)
    tiles_e = (counts + T - 1) // T
    pad_sz = tiles_e * T
    pad_off = jnp.cumsum(pad_sz) - pad_sz
    oh = (iflat[:, None] == jnp.arange(E, dtype=i32)[None, :])
    base = jnp.dot(oh.astype(jnp.float32), pad_off.astype(jnp.float32),
                   precision=lax.Precision.HIGHEST)
    slot = base.astype(i32) + rank
    total_tiles = jnp.sum(tiles_e)
    tile_cum = jnp.cumsum(tiles_e)
    gidx = jnp.arange(n_tiles_max, dtype=i32)
    tile_expert = jnp.minimum(
        jnp.searchsorted(tile_cum, gidx, side="right").astype(i32), E - 1)
    tile_valid = (gidx < total_tiles).astype(i32)
    tile_blk = jnp.where(tile_valid == 1, gidx, total_tiles - 1).astype(i32)
    n = itop.shape[0]
    slot_km = slot.reshape(n, K).T.reshape(-1)
    return slot_km, tile_expert, tile_blk, tile_valid


def _sc_scatter_dispatch(src, idx, n_rows_out, chunk):
    m = idx.shape[0]
    n, d = src.shape
    per_w = m // _NW
    n_iter = per_w // chunk
    planes_per_w = n // per_w
    mesh = plsc.VectorSubcoreMesh(core_axis_name="c", subcore_axis_name="s")

    @functools.partial(
        pl.kernel,
        out_type=jax.ShapeDtypeStruct((n_rows_out, d), src.dtype),
        mesh=mesh,
        scratch_types=[
            pltpu.VMEM((chunk,), jnp.int32),
            pltpu.VMEM((chunk, d), src.dtype),
            pltpu.SemaphoreType.DMA,
        ],
    )
    def scatter_k(src_hbm, idx_hbm, out_hbm, idx_v, rows_v, sem):
        wid = lax.axis_index("s") * _NC + lax.axis_index("c")
        base = wid * per_w
        t_base = (wid % planes_per_w) * per_w
        for i in range(n_iter):
            pltpu.sync_copy(idx_hbm.at[pl.ds(base + i * chunk, chunk)], idx_v)
            pltpu.sync_copy(src_hbm.at[pl.ds(t_base + i * chunk, chunk)], rows_v)
            pltpu.async_copy(rows_v, out_hbm.at[idx_v], sem).wait()

    return scatter_k(src, idx)


def _sc_gather(src, idx, chunk):
    m = idx.shape[0]
    d = src.shape[1]
    per_w = m // _NW
    n_iter = per_w // chunk
    mesh = plsc.VectorSubcoreMesh(core_axis_name="c", subcore_axis_name="s")

    @functools.partial(
        pl.kernel,
        out_type=jax.ShapeDtypeStruct((m, d), src.dtype),
        mesh=mesh,
        scratch_types=[
            pltpu.VMEM((chunk,), jnp.int32),
            pltpu.VMEM((chunk, d), src.dtype),
            pltpu.SemaphoreType.DMA,
        ],
    )
    def gather_k(src_hbm, idx_hbm, out_hbm, idx_v, rows_v, sem):
        wid = lax.axis_index("s") * _NC + lax.axis_index("c")
        base = wid * per_w
        for i in range(n_iter):
            off = base + i * chunk
            pltpu.sync_copy(idx_hbm.at[pl.ds(off, chunk)], idx_v)
            pltpu.async_copy(src_hbm.at[idx_v], rows_v, sem).wait()
            pltpu.sync_copy(rows_v, out_hbm.at[pl.ds(off, chunk)])

    return gather_k(src, idx)


def _ffn_body(blk_ref, exp_ref, val_ref, xg_ref, w1_ref, b1_ref, w2_ref,
              b2_ref, out_ref):
    g = pl.program_id(0)

    @pl.when(val_ref[g] == 1)
    def _():
        x = xg_ref[...]
        h = lax.dot_general(
            x, w1_ref[0], (((1,), (1,)), ((), ())),
            preferred_element_type=jnp.float32)
        h = h + b1_ref[0]
        h = 0.5 * h * (1.0 + lax.erf(h * 0.7071067811865476))
        y = lax.dot_general(
            h, w2_ref[0], (((1,), (1,)), ((), ())),
            preferred_element_type=jnp.float32)
        out_ref[...] = y + b2_ref[0]


def _grouped_ffn(xg, W1, b1, W2, b2, tile_blk, tile_expert, tile_valid,
                 n_tiles_max):
    d = W1.shape[2]
    h = W1.shape[1]
    n_rows = n_tiles_max * T
    grid_spec = pltpu.PrefetchScalarGridSpec(
        num_scalar_prefetch=3,
        grid=(n_tiles_max,),
        in_specs=[
            pl.BlockSpec((T, d), lambda g, blk, exp, val: (blk[g], 0)),
            pl.BlockSpec((1, h, d), lambda g, blk, exp, val: (exp[g], 0, 0)),
            pl.BlockSpec((1, 1, h), lambda g, blk, exp, val: (exp[g], 0, 0)),
            pl.BlockSpec((1, d, h), lambda g, blk, exp, val: (exp[g], 0, 0)),
            pl.BlockSpec((1, 1, d), lambda g, blk, exp, val: (exp[g], 0, 0)),
        ],
        out_specs=pl.BlockSpec((T, d), lambda g, blk, exp, val: (blk[g], 0)),
    )
    return pl.pallas_call(
        _ffn_body,
        grid_spec=grid_spec,
        out_shape=jax.ShapeDtypeStruct((n_rows, d), jnp.float32),
    )(tile_blk, tile_expert, tile_valid, xg, W1, b1.reshape(E, 1, h), W2,
      b2.reshape(E, 1, d))


def _combine_body(y0_ref, y1_ref, y2_ref, y3_ref, w_ref, out_ref):
    w = w_ref[...]
    acc = y0_ref[...] * w[:, 0:1]
    acc = acc + y1_ref[...] * w[:, 1:2]
    acc = acc + y2_ref[...] * w[:, 2:3]
    out_ref[...] = acc + y3_ref[...] * w[:, 3:4]


def _combine(yp, wtop, n, d):
    blk = 512
    nb = n // blk
    plane_specs = [
        pl.BlockSpec((blk, d), (lambda i, kk=kk: (i + kk * nb, 0)))
        for kk in range(K)
    ]
    return pl.pallas_call(
        _combine_body,
        grid=(nb,),
        in_specs=plane_specs + [pl.BlockSpec((blk, K), lambda i: (i, 0))],
        out_specs=pl.BlockSpec((blk, d), lambda i: (i, 0)),
        out_shape=jax.ShapeDtypeStruct((n, d), jnp.float32),
    )(yp, yp, yp, yp, wtop)


def kernel(x, Wg, bg, W1, b1, W2, b2):
    b, s, d = x.shape
    n = b * s
    nk = n * K
    n_tiles_max = nk // T + E
    x2d = x.reshape(n, d)

    logits, wtop, itop, rank, counts = _routing(x2d, Wg, bg)
    slot_km, tile_expert, tile_blk, tile_valid = (
        _dispatch_metadata(itop, rank, counts, n_tiles_max))

    n_rows = n_tiles_max * T
    xg = _sc_scatter_dispatch(x2d, slot_km, n_rows, chunk=128)
    yg = _grouped_ffn(xg, W1, b1, W2, b2, tile_blk, tile_expert,
                      tile_valid, n_tiles_max)
    yp = _sc_gather(yg, slot_km, chunk=128)
    final = _combine(yp, wtop, n, d)
    return final.reshape(b, s, d), logits
